# Initial kernel scaffold; baseline (speedup 1.0000x reference)
import jax
import jax.numpy as jnp
from jax.experimental import pallas as pl


def kernel(x, edge_index, edge_attr, batch, Wm, We, Ws, Wu, Pm, Pe, Ps, Pu, W1, b1, W2, b2, W3, b3):
    raise NotImplementedError("write your pallas kernel here")


# speedup vs baseline: 1.6365x; 1.6365x over previous
import functools

import jax
import jax.numpy as jnp
from jax import lax
from jax.experimental import pallas as pl
from jax.experimental.pallas import tpu as pltpu
from jax.experimental.pallas import tpu_sc as plsc

N = 10000
E = 160000
D = 256
DH = 128
G = 64

NB = 400
EB = 2000
NSUB = 16
EPT = E // NSUB
CE = 80
NCH = EPT // CE
NPAD = 10240
RPT = NPAD // NSUB
ZR = 128

_HI = jax.lax.Precision.HIGHEST


def _dot(a, b):
    return jnp.dot(a, b, preferred_element_type=jnp.float32)


def _pre_h_body(x_ref, wm_ref, pm_ref, rl_ref, rh_ref, fl_ref, fh_ref):
    xb = x_ref[...]
    hr = _dot(xb, wm_ref[...])
    rl_ref[...] = hr[:, :DH]
    rh_ref[...] = hr[:, DH:]
    hf = _dot(xb, pm_ref[...])
    fl_ref[...] = hf[:, :DH]
    fh_ref[...] = hf[:, DH:]


def _pre_h(x, Wm, Pm):
    n_blocks = N // NB
    return pl.pallas_call(
        _pre_h_body,
        grid=(n_blocks,),
        in_specs=[
            pl.BlockSpec((NB, D), lambda i: (i, 0)),
            pl.BlockSpec((D, D), lambda i: (0, 0)),
            pl.BlockSpec((D, D), lambda i: (0, 0)),
        ],
        out_specs=[pl.BlockSpec((NB, DH), lambda i: (i, 0))] * 4,
        out_shape=[jax.ShapeDtypeStruct((N, DH), jnp.float32)] * 4,
    )(x, Wm, Pm)


def _pre_ew_body(ea_ref, we_ref, pe_ref, rl_ref, rh_ref, fl_ref, fh_ref):
    eb = ea_ref[...]
    er = _dot(eb, we_ref[...])
    rl_ref[...] = er[:, :DH]
    rh_ref[...] = er[:, DH:]
    ef = _dot(eb, pe_ref[...])
    fl_ref[...] = ef[:, :DH]
    fh_ref[...] = ef[:, DH:]


def _pre_ew(edge_attr, We, Pe):
    de = edge_attr.shape[1]
    n_blocks = E // EB
    return pl.pallas_call(
        _pre_ew_body,
        grid=(n_blocks,),
        in_specs=[
            pl.BlockSpec((EB, de), lambda i: (i, 0)),
            pl.BlockSpec((de, D), lambda i: (0, 0)),
            pl.BlockSpec((de, D), lambda i: (0, 0)),
        ],
        out_specs=[pl.BlockSpec((EB, DH), lambda i: (i, 0))] * 4,
        out_shape=[jax.ShapeDtypeStruct((E, DH), jnp.float32)] * 4,
    )(edge_attr, We, Pe)


def _sc_body(src_hbm, dst_hbm, hrl, hrh, hfl, hfh, ewrl, ewrh, ewfl, ewfh,
             orl, orh, ofl, ofh,
             gat_v, ew_v, src_v, dst_v, zbuf, acc, sem):
    cid = lax.axis_index("c")
    sid = lax.axis_index("s")
    ebase = sid * EPT
    rbase = sid * RPT

    def _zrow(i, _):
        for j in range(DH // 16):
            zbuf[i, pl.ds(j * 16, 16)] = jnp.zeros((16,), jnp.float32)
        return 0

    lax.fori_loop(0, zbuf.shape[0], _zrow, 0)

    def process(h_hbm, ew_hbm, out_hbm):
        for k in range(RPT // zbuf.shape[0]):
            pltpu.sync_copy(zbuf, acc.at[pl.ds(rbase + k * zbuf.shape[0],
                                               zbuf.shape[0])])
        plsc.subcore_barrier()

        def chunk(c, _):
            e0 = ebase + c * CE
            pltpu.sync_copy(src_hbm.at[pl.ds(e0, CE)], src_v)
            pltpu.sync_copy(dst_hbm.at[pl.ds(e0, CE)], dst_v)
            pltpu.async_copy(h_hbm.at[src_v], gat_v, sem).wait()
            pltpu.sync_copy(ew_hbm.at[pl.ds(e0, CE)], ew_v)

            def row(i, _):
                for j in range(DH // 16):
                    s = pl.ds(j * 16, 16)
                    gat_v[i, s] = jnp.maximum(gat_v[i, s] + ew_v[i, s], 0.0)
                return 0

            lax.fori_loop(0, CE, row, 0)
            pltpu.sync_copy(gat_v, acc.at[dst_v], add=True)
            return 0

        lax.fori_loop(0, NCH, chunk, 0)
        plsc.subcore_barrier()
        pltpu.sync_copy(acc.at[pl.ds(rbase, RPT)],
                        out_hbm.at[pl.ds(rbase, RPT)])
        plsc.subcore_barrier()

    @pl.when(cid == 0)
    def _():
        process(hrl, ewrl, orl)
        process(hfl, ewfl, ofl)

    @pl.when(cid == 1)
    def _():
        process(hrh, ewrh, orh)
        process(hfh, ewfh, ofh)


def _sc_edge(src, dst, h4, ew4):
    f = pl.kernel(
        _sc_body,
        out_type=[jax.ShapeDtypeStruct((NPAD, DH), jnp.float32)] * 4,
        mesh=plsc.VectorSubcoreMesh(core_axis_name="c", subcore_axis_name="s",
                                    num_cores=2, num_subcores=NSUB),
        scratch_types=[
            pltpu.VMEM((CE, DH), jnp.float32),
            pltpu.VMEM((CE, DH), jnp.float32),
            pltpu.VMEM((CE,), jnp.int32),
            pltpu.VMEM((CE,), jnp.int32),
            pltpu.VMEM((ZR, DH), jnp.float32),
            pltpu.VMEM_SHARED((NPAD, DH), jnp.float32),
            pltpu.SemaphoreType.DMA,
        ],
    )
    return f(src, dst, *h4, *ew4)


def _post_body(x_ref, arl_ref, arh_ref, afl_ref, afh_ref, b_ref,
               ws_ref, wu_ref, ps_ref, pu_ref,
               w1_ref, b1_ref, w2_ref, b2_ref, w3_ref, b3_ref,
               feats_ref, gr_ref, sc_ref, sums, cnts):
    i = pl.program_id(0)

    @pl.when(i == 0)
    def _():
        sums[...] = jnp.zeros_like(sums)
        cnts[...] = jnp.zeros_like(cnts)

    xb = x_ref[...]
    wu = wu_ref[...]
    rep = jnp.maximum(
        _dot(arl_ref[...], wu[:DH, :]) + _dot(arh_ref[...], wu[DH:, :])
        + _dot(xb, ws_ref[...]), 0.0)
    pu = pu_ref[...]
    feats_ref[...] = jnp.maximum(
        _dot(afl_ref[...], pu[:DH, :]) + _dot(afh_ref[...], pu[DH:, :])
        + _dot(xb, ps_ref[...]), 0.0)

    bvals = b_ref[0, 0, :]
    onehot = (bvals[:, None]
              == jax.lax.broadcasted_iota(jnp.int32, (NB, G), 1)
              ).astype(jnp.float32)
    sums[...] += jax.lax.dot_general(
        onehot, rep, dimension_numbers=(((0,), (0,)), ((), ())),
        preferred_element_type=jnp.float32, precision=_HI)
    cnts[...] += jnp.broadcast_to(jnp.sum(onehot, axis=0)[:, None], (G, D))

    @pl.when(i == pl.num_programs(0) - 1)
    def _():
        gr = sums[...] / jnp.maximum(cnts[...], 1.0)
        gr_ref[...] = gr
        h1 = jnp.maximum(_dot(gr, w1_ref[...]) + b1_ref[...], 0.0)
        h2 = jnp.maximum(_dot(h1, w2_ref[...]) + b2_ref[...], 0.0)
        sc_ref[...] = _dot(h2, w3_ref[...])[:, :1] + b3_ref[...]


def _post(x, aggs, batch3, Ws, Wu, Ps, Pu, W1, b1, W2, b2, w3row, b3):
    n_blocks = N // NB
    full = lambda r, c: pl.BlockSpec((r, c), lambda i: (0, 0))
    return pl.pallas_call(
        _post_body,
        grid=(n_blocks,),
        in_specs=[
            pl.BlockSpec((NB, D), lambda i: (i, 0)),
            pl.BlockSpec((NB, DH), lambda i: (i, 0)),
            pl.BlockSpec((NB, DH), lambda i: (i, 0)),
            pl.BlockSpec((NB, DH), lambda i: (i, 0)),
            pl.BlockSpec((NB, DH), lambda i: (i, 0)),
            pl.BlockSpec((1, 1, NB), lambda i: (i, 0, 0)),
            full(D, D), full(D, D), full(D, D), full(D, D),
            full(D, D), full(1, D), full(D, D), full(1, D),
            full(D, DH), full(1, 1),
        ],
        out_specs=[
            pl.BlockSpec((NB, D), lambda i: (i, 0)),
            pl.BlockSpec((G, D), lambda i: (0, 0)),
            pl.BlockSpec((G, 1), lambda i: (0, 0)),
        ],
        out_shape=[
            jax.ShapeDtypeStruct((N, D), jnp.float32),
            jax.ShapeDtypeStruct((G, D), jnp.float32),
            jax.ShapeDtypeStruct((G, 1), jnp.float32),
        ],
        scratch_shapes=[
            pltpu.VMEM((G, D), jnp.float32),
            pltpu.VMEM((G, D), jnp.float32),
        ],
    )(x, *aggs, batch3, Ws, Wu, Ps, Pu, W1, b1, W2, b2, w3row, b3)


def kernel(x, edge_index, edge_attr, batch,
           Wm, We, Ws, Wu, Pm, Pe, Ps, Pu, W1, b1, W2, b2, W3, b3):
    src = edge_index[0]
    dst = edge_index[1]
    h4 = _pre_h(x, Wm, Pm)
    ew4 = _pre_ew(edge_attr, We, Pe)
    aggs = _sc_edge(src, dst, h4, ew4)
    batch3 = batch.reshape(N // NB, 1, NB)
    w3pad = jnp.pad(W3, ((0, 0), (0, DH - W3.shape[1])))
    feats, gr, sc = _post(
        x, aggs, batch3, Ws, Wu, Ps, Pu,
        W1, b1.reshape(1, D), W2, b2.reshape(1, D),
        w3pad, b3.reshape(1, 1))
    return (sc[:, 0], gr, feats)


# speedup vs baseline: 3.0458x; 1.8611x over previous
import functools

import jax
import jax.numpy as jnp
from jax import lax
from jax.experimental import pallas as pl
from jax.experimental.pallas import tpu as pltpu
from jax.experimental.pallas import tpu_sc as plsc

N = 10000
E = 160000
D = 256
DH = 128
G = 64

NB = 400
EB = 2000
NSUB = 16
EPT = E // NSUB
CE = 40
NCH = EPT // CE
NPAD = 10240
RPT = NPAD // NSUB

_HI = jax.lax.Precision.HIGHEST


def _dot(a, b):
    return jnp.dot(a, b, preferred_element_type=jnp.float32)


def _pre_h_body(x_ref, wm_ref, pm_ref, rl_ref, rh_ref, fl_ref, fh_ref):
    xb = x_ref[...]
    hr = _dot(xb, wm_ref[...])
    rl_ref[...] = hr[:, :DH]
    rh_ref[...] = hr[:, DH:]
    hf = _dot(xb, pm_ref[...])
    fl_ref[...] = hf[:, :DH]
    fh_ref[...] = hf[:, DH:]


def _pre_h(x, Wm, Pm):
    n_blocks = N // NB
    return pl.pallas_call(
        _pre_h_body,
        grid=(n_blocks,),
        in_specs=[
            pl.BlockSpec((NB, D), lambda i: (i, 0)),
            pl.BlockSpec((D, D), lambda i: (0, 0)),
            pl.BlockSpec((D, D), lambda i: (0, 0)),
        ],
        out_specs=[pl.BlockSpec((NB, DH), lambda i: (i, 0))] * 4,
        out_shape=[jax.ShapeDtypeStruct((N, DH), jnp.float32)] * 4,
    )(x, Wm, Pm)


def _pre_ew_body(ea_ref, we_ref, pe_ref, rl_ref, rh_ref, fl_ref, fh_ref):
    eb = ea_ref[...]
    er = _dot(eb, we_ref[...])
    rl_ref[...] = er[:, :DH]
    rh_ref[...] = er[:, DH:]
    ef = _dot(eb, pe_ref[...])
    fl_ref[...] = ef[:, :DH]
    fh_ref[...] = ef[:, DH:]


def _pre_ew(edge_attr, We, Pe):
    de = edge_attr.shape[1]
    n_blocks = E // EB
    return pl.pallas_call(
        _pre_ew_body,
        grid=(n_blocks,),
        in_specs=[
            pl.BlockSpec((EB, de), lambda i: (i, 0)),
            pl.BlockSpec((de, D), lambda i: (0, 0)),
            pl.BlockSpec((de, D), lambda i: (0, 0)),
        ],
        out_specs=[pl.BlockSpec((EB, DH), lambda i: (i, 0))] * 4,
        out_shape=[jax.ShapeDtypeStruct((E, DH), jnp.float32)] * 4,
    )(edge_attr, We, Pe)


NBUF = 3
NMAIN = (NCH - 1) // NBUF * NBUF


def _sc_body(src_hbm, dst_hbm, zero_hbm,
             hrl, hrh, hfl, hfh, ewrl, ewrh, ewfl, ewfh,
             orl, orh, ofl, ofh,
             gatb, ewb, sv0, sv1, sv2, dv0, dv1, dv2,
             acc,
             src_sems, dst_sems, gat_sems, ew_sems, sc_sems):
    srcv = [sv0, sv1, sv2]
    dstv = [dv0, dv1, dv2]
    cid = lax.axis_index("c")
    sid = lax.axis_index("s")
    ebase = sid * EPT
    rbase = sid * RPT

    def process(h_hbm, ew_hbm, out_hbm):
        def start_idx(c, b):
            e0 = ebase + c * CE
            pltpu.async_copy(src_hbm.at[pl.ds(e0, CE)], srcv[b],
                             src_sems.at[b])
            pltpu.async_copy(dst_hbm.at[pl.ds(e0, CE)], dstv[b],
                             dst_sems.at[b])

        def wait_idx(c, b):
            e0 = ebase + c * CE
            pltpu.make_async_copy(src_hbm.at[pl.ds(e0, CE)], srcv[b],
                                  src_sems.at[b]).wait()
            pltpu.make_async_copy(dst_hbm.at[pl.ds(e0, CE)], dstv[b],
                                  dst_sems.at[b]).wait()

        def start_gat(c, b):
            pltpu.async_copy(h_hbm.at[srcv[b]], gatb.at[b], gat_sems.at[b])
            pltpu.async_copy(ew_hbm.at[pl.ds(ebase + c * CE, CE)], ewb.at[b],
                             ew_sems.at[b])

        def wait_gat(c, b):
            pltpu.make_async_copy(h_hbm.at[srcv[b]], gatb.at[b],
                                  gat_sems.at[b]).wait()
            pltpu.make_async_copy(ew_hbm.at[pl.ds(ebase + c * CE, CE)],
                                  ewb.at[b], ew_sems.at[b]).wait()

        def start_sc(b):
            pltpu.async_copy(gatb.at[b], acc.at[dstv[b]],
                             sc_sems.at[b], add=True)

        def wait_sc(b):
            pltpu.make_async_copy(gatb.at[b], acc.at[dstv[b]],
                                  sc_sems.at[b]).wait()

        def compute(b):
            def row(i, _):
                for j in range(DH // 16):
                    s = pl.ds(j * 16, 16)
                    gatb[b, i, s] = jnp.maximum(
                        gatb[b, i, s] + ewb[b, i, s], 0.0)
                return 0

            lax.fori_loop(0, CE, row, 0)

        pltpu.sync_copy(zero_hbm.at[pl.ds(rbase, RPT)],
                        acc.at[pl.ds(rbase, RPT)])
        plsc.subcore_barrier()

        start_idx(0, 0)
        start_idx(1, 1)
        wait_idx(0, 0)
        start_gat(0, 0)

        def group(g, _):
            for b in range(NBUF):
                c = g * NBUF + b
                b2 = (b + 2) % NBUF

                @pl.when(jnp.logical_and(c + 2 >= NBUF, c + 2 < NCH))
                def _():
                    wait_sc(b2)

                @pl.when(c + 2 < NCH)
                def _():
                    start_idx(c + 2, b2)

                b1 = (b + 1) % NBUF
                wait_idx(c + 1, b1)
                start_gat(c + 1, b1)

                wait_gat(c, b)
                compute(b)
                start_sc(b)
            return 0

        lax.fori_loop(0, NMAIN // NBUF, group, 0)
        for c in range(NMAIN, NCH):
            b = c % NBUF
            wait_gat(c, b)
            compute(b)
            start_sc(b)
        for b in range(NBUF):
            wait_sc(b)
        plsc.subcore_barrier()
        pltpu.sync_copy(acc.at[pl.ds(rbase, RPT)],
                        out_hbm.at[pl.ds(rbase, RPT)])
        plsc.subcore_barrier()

    @pl.when(cid == 0)
    def _():
        process(hrl, ewrl, orl)
        process(hfl, ewfl, ofl)

    @pl.when(cid == 1)
    def _():
        process(hrh, ewrh, orh)
        process(hfh, ewfh, ofh)


def _sc_edge(edge_index, h4, ew4):
    f = pl.kernel(
        _sc_body,
        out_type=[jax.ShapeDtypeStruct((NPAD, DH), jnp.float32)] * 4,
        mesh=plsc.VectorSubcoreMesh(core_axis_name="c", subcore_axis_name="s",
                                    num_cores=2, num_subcores=NSUB),
        scratch_types=[
            pltpu.VMEM((NBUF, CE, DH), jnp.float32),
            pltpu.VMEM((NBUF, CE, DH), jnp.float32),
        ] + [pltpu.VMEM((CE,), jnp.int32)] * (2 * NBUF) + [
            pltpu.VMEM_SHARED((NPAD, DH), jnp.float32),
            pltpu.SemaphoreType.DMA((NBUF,)),
            pltpu.SemaphoreType.DMA((NBUF,)),
            pltpu.SemaphoreType.DMA((NBUF,)),
            pltpu.SemaphoreType.DMA((NBUF,)),
            pltpu.SemaphoreType.DMA((NBUF,)),
        ],
    )
    zero = jnp.zeros((NPAD, DH), jnp.float32)
    return f(edge_index[0], edge_index[1], zero, *h4, *ew4)


def _post_body(x_ref, arl_ref, arh_ref, afl_ref, afh_ref, b_ref,
               ws_ref, wu_ref, ps_ref, pu_ref,
               w1_ref, b1_ref, w2_ref, b2_ref, w3_ref, b3_ref,
               feats_ref, gr_ref, sc_ref, sums, cnts):
    i = pl.program_id(0)

    @pl.when(i == 0)
    def _():
        sums[...] = jnp.zeros_like(sums)
        cnts[...] = jnp.zeros_like(cnts)

    xb = x_ref[...]
    wu = wu_ref[...]
    rep = jnp.maximum(
        _dot(arl_ref[...], wu[:DH, :]) + _dot(arh_ref[...], wu[DH:, :])
        + _dot(xb, ws_ref[...]), 0.0)
    pu = pu_ref[...]
    feats_ref[...] = jnp.maximum(
        _dot(afl_ref[...], pu[:DH, :]) + _dot(afh_ref[...], pu[DH:, :])
        + _dot(xb, ps_ref[...]), 0.0)

    bvals = b_ref[0, 0, :]
    onehot = (bvals[:, None]
              == jax.lax.broadcasted_iota(jnp.int32, (NB, G), 1)
              ).astype(jnp.float32)
    sums[...] += jax.lax.dot_general(
        onehot, rep, dimension_numbers=(((0,), (0,)), ((), ())),
        preferred_element_type=jnp.float32, precision=_HI)
    cnts[...] += jnp.broadcast_to(jnp.sum(onehot, axis=0)[:, None], (G, D))

    @pl.when(i == pl.num_programs(0) - 1)
    def _():
        gr = sums[...] / jnp.maximum(cnts[...], 1.0)
        gr_ref[...] = gr
        h1 = jnp.maximum(_dot(gr, w1_ref[...]) + b1_ref[...], 0.0)
        h2 = jnp.maximum(_dot(h1, w2_ref[...]) + b2_ref[...], 0.0)
        sc_ref[...] = _dot(h2, w3_ref[...])[:, :1] + b3_ref[...]


def _post(x, aggs, batch3, Ws, Wu, Ps, Pu, W1, b1, W2, b2, w3row, b3):
    n_blocks = N // NB
    full = lambda r, c: pl.BlockSpec((r, c), lambda i: (0, 0))
    return pl.pallas_call(
        _post_body,
        grid=(n_blocks,),
        in_specs=[
            pl.BlockSpec((NB, D), lambda i: (i, 0)),
            pl.BlockSpec((NB, DH), lambda i: (i, 0)),
            pl.BlockSpec((NB, DH), lambda i: (i, 0)),
            pl.BlockSpec((NB, DH), lambda i: (i, 0)),
            pl.BlockSpec((NB, DH), lambda i: (i, 0)),
            pl.BlockSpec((1, 1, NB), lambda i: (i, 0, 0)),
            full(D, D), full(D, D), full(D, D), full(D, D),
            full(D, D), full(1, D), full(D, D), full(1, D),
            full(D, DH), full(1, 1),
        ],
        out_specs=[
            pl.BlockSpec((NB, D), lambda i: (i, 0)),
            pl.BlockSpec((G, D), lambda i: (0, 0)),
            pl.BlockSpec((G, 1), lambda i: (0, 0)),
        ],
        out_shape=[
            jax.ShapeDtypeStruct((N, D), jnp.float32),
            jax.ShapeDtypeStruct((G, D), jnp.float32),
            jax.ShapeDtypeStruct((G, 1), jnp.float32),
        ],
        scratch_shapes=[
            pltpu.VMEM((G, D), jnp.float32),
            pltpu.VMEM((G, D), jnp.float32),
        ],
    )(x, *aggs, batch3, Ws, Wu, Ps, Pu, W1, b1, W2, b2, w3row, b3)


def kernel(x, edge_index, edge_attr, batch,
           Wm, We, Ws, Wu, Pm, Pe, Ps, Pu, W1, b1, W2, b2, W3, b3):
    h4 = _pre_h(x, Wm, Pm)
    ew4 = _pre_ew(edge_attr, We, Pe)
    aggs = _sc_edge(edge_index, h4, ew4)
    batch3 = batch.reshape(N // NB, 1, NB)
    w3pad = jnp.pad(W3, ((0, 0), (0, DH - W3.shape[1])))
    feats, gr, sc = _post(
        x, aggs, batch3, Ws, Wu, Ps, Pu,
        W1, b1.reshape(1, D), W2, b2.reshape(1, D),
        w3pad, b3.reshape(1, 1))
    return (sc[:, 0], gr, feats)


# speedup vs baseline: 3.1549x; 1.0358x over previous
import functools

import jax
import jax.numpy as jnp
from jax import lax
from jax.experimental import pallas as pl
from jax.experimental.pallas import tpu as pltpu
from jax.experimental.pallas import tpu_sc as plsc

N = 10000
E = 160000
D = 256
DH = 128
G = 64

NB = 400
EB = 2000
NSUB = 16
EPT = E // NSUB
CE = 40
NCH = EPT // CE
NPAD = 10240
RPT = NPAD // NSUB

_HI = jax.lax.Precision.HIGHEST


def _dot(a, b):
    return jnp.dot(a, b, preferred_element_type=jnp.float32)


def _pre_body(x_ref, ea_ref, wm_ref, pm_ref, we_ref, pe_ref,
              hrl_ref, hrh_ref, hfl_ref, hfh_ref,
              erl_ref, erh_ref, efl_ref, efh_ref):
    i = pl.program_id(0)

    @pl.when(i < N // NB)
    def _():
        xb = x_ref[...]
        hr = _dot(xb, wm_ref[...])
        hrl_ref[...] = hr[:, :DH]
        hrh_ref[...] = hr[:, DH:]
        hf = _dot(xb, pm_ref[...])
        hfl_ref[...] = hf[:, :DH]
        hfh_ref[...] = hf[:, DH:]

    eb = ea_ref[...]
    er = _dot(eb, we_ref[...])
    erl_ref[...] = er[:, :DH]
    erh_ref[...] = er[:, DH:]
    ef = _dot(eb, pe_ref[...])
    efl_ref[...] = ef[:, :DH]
    efh_ref[...] = ef[:, DH:]


def _pre(x, edge_attr, Wm, Pm, We, Pe):
    de = edge_attr.shape[1]
    nh = N // NB
    full = lambda r, c: pl.BlockSpec((r, c), lambda i: (0, 0))
    hmap = lambda i: (jnp.minimum(i, nh - 1), 0)
    return pl.pallas_call(
        _pre_body,
        grid=(E // EB,),
        in_specs=[
            pl.BlockSpec((NB, D), hmap),
            pl.BlockSpec((EB, de), lambda i: (i, 0)),
            full(D, D), full(D, D), full(de, D), full(de, D),
        ],
        out_specs=[pl.BlockSpec((NB, DH), hmap)] * 4
        + [pl.BlockSpec((EB, DH), lambda i: (i, 0))] * 4,
        out_shape=[jax.ShapeDtypeStruct((N, DH), jnp.float32)] * 4
        + [jax.ShapeDtypeStruct((E, DH), jnp.float32)] * 4,
    )(x, edge_attr, Wm, Pm, We, Pe)


NBUF = 3
NMAIN = (NCH - 1) // NBUF * NBUF


def _sc_body(src_hbm, dst_hbm,
             hrl, hrh, hfl, hfh, ewrl, ewrh, ewfl, ewfh,
             orl, orh, ofl, ofh,
             gatb, ewb, sv0, sv1, sv2, dv0, dv1, dv2,
             acc,
             src_sems, dst_sems, gat_sems, ew_sems, sc_sems):
    srcv = [sv0, sv1, sv2]
    dstv = [dv0, dv1, dv2]
    cid = lax.axis_index("c")
    sid = lax.axis_index("s")
    ebase = sid * EPT
    rbase = sid * RPT

    def process(h_hbm, ew_hbm, out_hbm):
        def start_idx(c, b):
            e0 = ebase + c * CE
            pltpu.async_copy(src_hbm.at[pl.ds(e0, CE)], srcv[b],
                             src_sems.at[b])
            pltpu.async_copy(dst_hbm.at[pl.ds(e0, CE)], dstv[b],
                             dst_sems.at[b])

        def wait_idx(c, b):
            e0 = ebase + c * CE
            pltpu.make_async_copy(src_hbm.at[pl.ds(e0, CE)], srcv[b],
                                  src_sems.at[b]).wait()
            pltpu.make_async_copy(dst_hbm.at[pl.ds(e0, CE)], dstv[b],
                                  dst_sems.at[b]).wait()

        def start_gat(c, b):
            pltpu.async_copy(h_hbm.at[srcv[b]], gatb.at[b], gat_sems.at[b])
            pltpu.async_copy(ew_hbm.at[pl.ds(ebase + c * CE, CE)], ewb.at[b],
                             ew_sems.at[b])

        def wait_gat(c, b):
            pltpu.make_async_copy(h_hbm.at[srcv[b]], gatb.at[b],
                                  gat_sems.at[b]).wait()
            pltpu.make_async_copy(ew_hbm.at[pl.ds(ebase + c * CE, CE)],
                                  ewb.at[b], ew_sems.at[b]).wait()

        def start_sc(b):
            pltpu.async_copy(gatb.at[b], acc.at[dstv[b]],
                             sc_sems.at[b], add=True)

        def wait_sc(b):
            pltpu.make_async_copy(gatb.at[b], acc.at[dstv[b]],
                                  sc_sems.at[b]).wait()

        def compute(b):
            def row(i, _):
                for j in range(DH // 16):
                    s = pl.ds(j * 16, 16)
                    gatb[b, i, s] = jnp.maximum(
                        gatb[b, i, s] + ewb[b, i, s], 0.0)
                return 0

            lax.fori_loop(0, CE, row, 0)

        def zrow(i, _):
            for j in range(DH // 16):
                gatb[0, i, pl.ds(j * 16, 16)] = jnp.zeros((16,), jnp.float32)
            return 0

        lax.fori_loop(0, CE, zrow, 0)
        for k in range(RPT // CE):
            pltpu.sync_copy(gatb.at[0], acc.at[pl.ds(rbase + k * CE, CE)])
        plsc.subcore_barrier()

        start_idx(0, 0)
        start_idx(1, 1)
        wait_idx(0, 0)
        start_gat(0, 0)

        def group(g, _):
            for b in range(NBUF):
                c = g * NBUF + b
                b2 = (b + 2) % NBUF

                @pl.when(jnp.logical_and(c + 2 >= NBUF, c + 2 < NCH))
                def _():
                    wait_sc(b2)

                @pl.when(c + 2 < NCH)
                def _():
                    start_idx(c + 2, b2)

                b1 = (b + 1) % NBUF
                wait_idx(c + 1, b1)
                start_gat(c + 1, b1)

                wait_gat(c, b)
                compute(b)
                start_sc(b)
            return 0

        lax.fori_loop(0, NMAIN // NBUF, group, 0)
        for c in range(NMAIN, NCH):
            b = c % NBUF
            wait_gat(c, b)
            compute(b)
            start_sc(b)
        for b in range(NBUF):
            wait_sc(b)
        plsc.subcore_barrier()
        pltpu.sync_copy(acc.at[pl.ds(rbase, RPT)],
                        out_hbm.at[pl.ds(rbase, RPT)])
        plsc.subcore_barrier()

    @pl.when(cid == 0)
    def _():
        process(hrl, ewrl, orl)
        process(hfl, ewfl, ofl)

    @pl.when(cid == 1)
    def _():
        process(hrh, ewrh, orh)
        process(hfh, ewfh, ofh)


def _sc_edge(edge_index, h4, ew4):
    f = pl.kernel(
        _sc_body,
        out_type=[jax.ShapeDtypeStruct((NPAD, DH), jnp.float32)] * 4,
        mesh=plsc.VectorSubcoreMesh(core_axis_name="c", subcore_axis_name="s",
                                    num_cores=2, num_subcores=NSUB),
        scratch_types=[
            pltpu.VMEM((NBUF, CE, DH), jnp.float32),
            pltpu.VMEM((NBUF, CE, DH), jnp.float32),
        ] + [pltpu.VMEM((CE,), jnp.int32)] * (2 * NBUF) + [
            pltpu.VMEM_SHARED((NPAD, DH), jnp.float32),
            pltpu.SemaphoreType.DMA((NBUF,)),
            pltpu.SemaphoreType.DMA((NBUF,)),
            pltpu.SemaphoreType.DMA((NBUF,)),
            pltpu.SemaphoreType.DMA((NBUF,)),
            pltpu.SemaphoreType.DMA((NBUF,)),
        ],
    )
    return f(edge_index[0], edge_index[1], *h4, *ew4)


def _post_body(x_ref, arl_ref, arh_ref, afl_ref, afh_ref, b_ref,
               ws_ref, wu_ref, ps_ref, pu_ref,
               w1_ref, b1_ref, w2_ref, b2_ref, w3_ref, b3_ref,
               feats_ref, gr_ref, sc_ref, sums, cnts):
    i = pl.program_id(0)

    @pl.when(i == 0)
    def _():
        sums[...] = jnp.zeros_like(sums)
        cnts[...] = jnp.zeros_like(cnts)

    xb = x_ref[...]
    wu = wu_ref[...]
    rep = jnp.maximum(
        _dot(arl_ref[...], wu[:DH, :]) + _dot(arh_ref[...], wu[DH:, :])
        + _dot(xb, ws_ref[...]), 0.0)
    pu = pu_ref[...]
    feats_ref[...] = jnp.maximum(
        _dot(afl_ref[...], pu[:DH, :]) + _dot(afh_ref[...], pu[DH:, :])
        + _dot(xb, ps_ref[...]), 0.0)

    bvals = b_ref[0, 0, :]
    onehot = (bvals[:, None]
              == jax.lax.broadcasted_iota(jnp.int32, (NB, G), 1)
              ).astype(jnp.float32)
    sums[...] += jax.lax.dot_general(
        onehot, rep, dimension_numbers=(((0,), (0,)), ((), ())),
        preferred_element_type=jnp.float32, precision=_HI)
    cnts[...] += jnp.broadcast_to(jnp.sum(onehot, axis=0)[:, None], (G, D))

    @pl.when(i == pl.num_programs(0) - 1)
    def _():
        gr = sums[...] / jnp.maximum(cnts[...], 1.0)
        gr_ref[...] = gr
        h1 = jnp.maximum(_dot(gr, w1_ref[...]) + b1_ref[...], 0.0)
        h2 = jnp.maximum(_dot(h1, w2_ref[...]) + b2_ref[...], 0.0)
        sc_ref[...] = _dot(h2, w3_ref[...])[:, :1] + b3_ref[...]


def _post(x, aggs, batch3, Ws, Wu, Ps, Pu, W1, b1, W2, b2, w3row, b3):
    n_blocks = N // NB
    full = lambda r, c: pl.BlockSpec((r, c), lambda i: (0, 0))
    return pl.pallas_call(
        _post_body,
        grid=(n_blocks,),
        in_specs=[
            pl.BlockSpec((NB, D), lambda i: (i, 0)),
            pl.BlockSpec((NB, DH), lambda i: (i, 0)),
            pl.BlockSpec((NB, DH), lambda i: (i, 0)),
            pl.BlockSpec((NB, DH), lambda i: (i, 0)),
            pl.BlockSpec((NB, DH), lambda i: (i, 0)),
            pl.BlockSpec((1, 1, NB), lambda i: (i, 0, 0)),
            full(D, D), full(D, D), full(D, D), full(D, D),
            full(D, D), full(1, D), full(D, D), full(1, D),
            full(D, DH), full(1, 1),
        ],
        out_specs=[
            pl.BlockSpec((NB, D), lambda i: (i, 0)),
            pl.BlockSpec((G, D), lambda i: (0, 0)),
            pl.BlockSpec((G, 1), lambda i: (0, 0)),
        ],
        out_shape=[
            jax.ShapeDtypeStruct((N, D), jnp.float32),
            jax.ShapeDtypeStruct((G, D), jnp.float32),
            jax.ShapeDtypeStruct((G, 1), jnp.float32),
        ],
        scratch_shapes=[
            pltpu.VMEM((G, D), jnp.float32),
            pltpu.VMEM((G, D), jnp.float32),
        ],
    )(x, *aggs, batch3, Ws, Wu, Ps, Pu, W1, b1, W2, b2, w3row, b3)


def kernel(x, edge_index, edge_attr, batch,
           Wm, We, Ws, Wu, Pm, Pe, Ps, Pu, W1, b1, W2, b2, W3, b3):
    pre8 = _pre(x, edge_attr, Wm, Pm, We, Pe)
    aggs = _sc_edge(edge_index, pre8[:4], pre8[4:])
    batch3 = batch.reshape(N // NB, 1, NB)
    w3pad = jnp.pad(W3, ((0, 0), (0, DH - W3.shape[1])))
    feats, gr, sc = _post(
        x, aggs, batch3, Ws, Wu, Ps, Pu,
        W1, b1.reshape(1, D), W2, b2.reshape(1, D),
        w3pad, b3.reshape(1, 1))
    return (sc[:, 0], gr, feats)
